# baseline (speedup 1.0000x reference)
import jax
import jax.numpy as jnp
from jax import lax
from jax.experimental import pallas as pl
from jax.experimental.pallas import tpu as pltpu

B = 16
HB = B // 2
L = 4096
TOKEN_DIM = 64
STRIDE = 56
NT = 73
OUT_W = NT * TOKEN_DIM
FULL_TILES = OUT_W // 128


def _tokenize_tc_body(in_hbm, out_hbm, in_v, flat_ref, tiled_ref, tr_ref,
                      sem_in, sem_out):
    cp_in = []
    for h in range(2):
        cp = pltpu.make_async_copy(
            in_hbm.at[pl.ds(HB * h, HB)], in_v.at[pl.ds(HB * h, HB)],
            sem_in,
        )
        cp.start()
        cp_in.append(cp)

    lane = lax.broadcasted_iota(jnp.int32, (HB, 128), 1)
    first_half = lane < TOKEN_DIM
    cp_out = []
    for h in range(2):
        rows = pl.ds(HB * h, HB)
        cp_in[h].wait()
        for k in range(FULL_TILES):
            a = in_v[rows, 112 * k:112 * k + 128]
            if k == 0:
                b = jnp.roll(a, 8, axis=1)
            else:
                b = in_v[rows, 112 * k - 8:112 * k + 120]
            flat_ref[rows, 128 * k:128 * k + 128] = jnp.where(
                first_half, a, b)
        flat_ref[rows, FULL_TILES * 128:] = in_v[rows, STRIDE * (NT - 1):]
        for w in range(NT // 8):
            tiled_ref[rows, 8 * w:8 * w + 8, :] = flat_ref[
                rows, 512 * w:512 * w + 512
            ].reshape(HB, 8, TOKEN_DIM)
        tiled_ref[rows, NT - 1:NT, :] = flat_ref[
            rows, OUT_W - TOKEN_DIM:
        ].reshape(HB, 1, TOKEN_DIM)
        tr_ref[rows] = jnp.transpose(tiled_ref[rows], (0, 2, 1))
        cp = pltpu.make_async_copy(
            tr_ref.at[rows], out_hbm.at[rows], sem_out)
        cp.start()
        cp_out.append(cp)
    cp_out[0].wait()
    cp_out[1].wait()


def kernel(inputs):
    t_out = pl.pallas_call(
        _tokenize_tc_body,
        out_shape=jax.ShapeDtypeStruct((B, TOKEN_DIM, NT), jnp.float32),
        in_specs=[pl.BlockSpec(memory_space=pl.ANY)],
        out_specs=pl.BlockSpec(memory_space=pl.ANY),
        scratch_shapes=[
            pltpu.VMEM((B, L), jnp.float32),
            pltpu.VMEM((B, OUT_W), jnp.float32),
            pltpu.VMEM((B, NT, TOKEN_DIM), jnp.float32),
            pltpu.VMEM((B, TOKEN_DIM, NT), jnp.float32),
            pltpu.SemaphoreType.DMA,
            pltpu.SemaphoreType.DMA,
        ],
    )(inputs)
    return t_out.transpose(0, 2, 1)


# speedup vs baseline: 1.0426x; 1.0426x over previous
import jax
import jax.numpy as jnp
from jax import lax
from jax.experimental import pallas as pl
from jax.experimental.pallas import tpu as pltpu

B = 16
L = 4096
TOKEN_DIM = 64
STRIDE = 56
NT = 73
OUT_W = NT * TOKEN_DIM
FULL_TILES = OUT_W // 128


def _tokenize_tc_body(in_ref, out_ref, flat_ref, tiled_ref, tr_ref, sem):
    lane = lax.broadcasted_iota(jnp.int32, (B, 128), 1)
    first_half = lane < TOKEN_DIM
    for k in range(FULL_TILES):
        a = in_ref[:, 112 * k:112 * k + 128]
        if k == 0:
            b = jnp.roll(a, 8, axis=1)
        else:
            b = in_ref[:, 112 * k - 8:112 * k + 120]
        flat_ref[:, 128 * k:128 * k + 128] = jnp.where(first_half, a, b)
    flat_ref[:, FULL_TILES * 128:] = in_ref[:, STRIDE * (NT - 1):]
    for w in range(NT // 8):
        tiled_ref[:, 8 * w:8 * w + 8, :] = flat_ref[
            :, 512 * w:512 * w + 512
        ].reshape(B, 8, TOKEN_DIM)
    tiled_ref[:, NT - 1:NT, :] = flat_ref[:, OUT_W - TOKEN_DIM:].reshape(
        B, 1, TOKEN_DIM
    )
    half = B // 2
    tr_ref[0:half] = jnp.transpose(tiled_ref[0:half], (0, 2, 1))
    cp0 = pltpu.make_async_copy(tr_ref.at[pl.ds(0, half)],
                                out_ref.at[pl.ds(0, half)], sem)
    cp0.start()
    tr_ref[half:B] = jnp.transpose(tiled_ref[half:B], (0, 2, 1))
    cp1 = pltpu.make_async_copy(tr_ref.at[pl.ds(half, half)],
                                out_ref.at[pl.ds(half, half)], sem)
    cp1.start()
    cp0.wait()
    cp1.wait()


def kernel(inputs):
    t_out = pl.pallas_call(
        _tokenize_tc_body,
        out_shape=jax.ShapeDtypeStruct((B, TOKEN_DIM, NT), jnp.float32),
        out_specs=pl.BlockSpec(memory_space=pl.ANY),
        scratch_shapes=[
            pltpu.VMEM((B, OUT_W), jnp.float32),
            pltpu.VMEM((B, NT, TOKEN_DIM), jnp.float32),
            pltpu.VMEM((B, TOKEN_DIM, NT), jnp.float32),
            pltpu.SemaphoreType.DMA,
        ],
    )(inputs)
    return t_out.transpose(0, 2, 1)
